# baseline (speedup 1.0000x reference)
import functools

import jax
import jax.numpy as jnp
from jax.experimental import pallas as pl
from jax.experimental.pallas import tpu as pltpu
from jax.experimental.pallas import tpu_sc as plsc

B, T, VOCAB, D_EMB, H, L_OUT, TAGS = 64, 128, 100000, 256, 512, 256, 50
G = 4 * H
N = T * B
UNROLL = 8
NB = T // UNROLL

_GATHER_WINDOW = 128


def _sc_gather(emb, idx):
    mesh = plsc.VectorSubcoreMesh(core_axis_name="core", subcore_axis_name="subcore")

    @pl.kernel(out_type=jax.ShapeDtypeStruct((N, D_EMB), emb.dtype), mesh=mesh)
    def gather_kernel(emb_hbm, i_hbm, o_hbm):
        def body(i_vmem, o_vmem):
            pltpu.sync_copy(emb_hbm.at[i_vmem.at[0]], o_vmem)

        pltpu.emit_pipeline(
            body,
            grid=(N // _GATHER_WINDOW,),
            in_specs=[pl.BlockSpec((1, _GATHER_WINDOW), index_map=lambda i: (0, i))],
            out_specs=[pl.BlockSpec((_GATHER_WINDOW, D_EMB), index_map=lambda i: (i, 0))],
            core_axis_name=("core", "subcore"),
            dimension_semantics=(pltpu.PARALLEL,),
        )(i_hbm, o_hbm)

    return gather_kernel(emb, idx.reshape(1, N))


def _dot(a, b):
    return jax.lax.dot_general(a, b, (((1,), (0,)), ((), ())),
                               preferred_element_type=jnp.float32)


def _dot_t(a, b):
    return jax.lax.dot_general(a, b, (((1,), (1,)), ((), ())),
                               preferred_element_type=jnp.float32)


def _recurrence_block(i, len_ref, sxf, sxb, store_f, store_b, hf, cf, hb, cb,
                      whf_ref, whb_ref):
    @pl.when(i == 1)
    def _():
        zero = jnp.zeros((B, H), jnp.float32)
        hf[...] = zero
        cf[...] = zero
        hb[...] = zero
        cb[...] = zero

    lens = len_ref[...]
    buf = (i - 1) % 2

    def gates(g, c):
        gi = jax.nn.sigmoid(g[:, :H])
        gf = jax.nn.sigmoid(g[:, H:2 * H])
        gg = jnp.tanh(g[:, 2 * H:3 * H])
        go = jax.nn.sigmoid(g[:, 3 * H:])
        c_new = gf * c + gi * gg
        h_new = go * jnp.tanh(c_new)
        return h_new, c_new

    t0 = (i - 1) * UNROLL
    for k in range(UNROLL):
        tf = t0 + k
        tb = T - 1 - tf
        g_f = sxf[buf, pl.ds(B * k, B), :] + _dot(hf[...].astype(jnp.bfloat16),
                                                  whf_ref[...])
        g_b = (sxb[buf, pl.ds(B * (UNROLL - 1 - k), B), :]
               + _dot(hb[...].astype(jnp.bfloat16), whb_ref[...]))
        hn_f, cn_f = gates(g_f, cf[...])
        hn_b, cn_b = gates(g_b, cb[...])
        m_f = lens > jnp.float32(tf)
        m_b = lens > jnp.float32(tb)
        zero = jnp.zeros((B, H), jnp.float32)
        store_f(k, tf, jnp.where(m_f, hn_f, zero))
        store_b(UNROLL - 1 - k, tb, jnp.where(m_b, hn_b, zero))
        hf[...] = jnp.where(m_f, hn_f, hf[...])
        cf[...] = jnp.where(m_f, cn_f, cf[...])
        hb[...] = jnp.where(m_b, hn_b, hb[...])
        cb[...] = jnp.where(m_b, cn_b, cb[...])


def _layer0_body(len_ref, pf_ref, pb_ref, wif_ref, wib_ref, whf_ref, whb_ref,
                 b_ref, yf_ref, yb_ref, sxf, sxb, hf, cf, hb, cb):
    i = pl.program_id(0)

    @pl.when(i < NB)
    def _():
        xf = pf_ref[...].reshape(UNROLL * B, D_EMB).astype(jnp.bfloat16)
        xb = pb_ref[...].reshape(UNROLL * B, D_EMB).astype(jnp.bfloat16)
        sxf[i % 2] = (_dot_t(xf, wif_ref[...]) + b_ref[:, :G]).astype(sxf.dtype)
        sxb[i % 2] = (_dot_t(xb, wib_ref[...]) + b_ref[:, G:]).astype(sxb.dtype)

    def store_f(k, tf, v):
        yf_ref[k] = v.astype(yf_ref.dtype)

    def store_b(kb, tb, v):
        yb_ref[kb] = v.astype(yb_ref.dtype)

    @pl.when(i > 0)
    def _():
        _recurrence_block(i, len_ref, sxf, sxb, store_f, store_b,
                          hf, cf, hb, cb, whf_ref, whb_ref)


def _layer1_body(len_ref, pfa_ref, pfb_ref, pba_ref, pbb_ref,
                 wif_ref, wib_ref, whf_ref, whb_ref, b_ref,
                 fcw_ref, fcb_ref, cls_ref, clsb_ref,
                 out1_ref, out2_ref, sxf, sxb, hf, cf, hb, cb, syf, syb):
    i = pl.program_id(0)

    @pl.when(i < NB)
    def _():
        ya_f = pfa_ref[...].reshape(UNROLL * B, H)
        yb_f = pfb_ref[...].reshape(UNROLL * B, H)
        ya_b = pba_ref[...].reshape(UNROLL * B, H)
        yb_b = pbb_ref[...].reshape(UNROLL * B, H)
        sxf[i % 2] = (_dot_t(ya_f, wif_ref[:, :H]) + _dot_t(yb_f, wif_ref[:, H:])
                      + b_ref[:, :G]).astype(sxf.dtype)
        sxb[i % 2] = (_dot_t(ya_b, wib_ref[:, :H]) + _dot_t(yb_b, wib_ref[:, H:])
                      + b_ref[:, G:]).astype(sxb.dtype)

    def store_f(k, tf, v):
        syf[tf] = v.astype(syf.dtype)

    def store_b(kb, tb, v):
        syb[tb] = v.astype(syb.dtype)

    @pl.when(i > 0)
    def _():
        _recurrence_block(i, len_ref, sxf, sxb, store_f, store_b,
                          hf, cf, hb, cb, whf_ref, whb_ref)

    @pl.when(i >= NB // 2 + 1)
    def _():
        def head_block(j, o_ref):
            ya = syf[pl.ds(j * UNROLL, UNROLL)].reshape(UNROLL * B, H)
            yb = syb[pl.ds(j * UNROLL, UNROLL)].reshape(UNROLL * B, H)
            h = (_dot_t(ya, fcw_ref[:, :H]) + _dot_t(yb, fcw_ref[:, H:])
                 + fcb_ref[...])
            h = jnp.where(h > 0, h, 0.01 * (jnp.exp(jnp.minimum(h, 0.0)) - 1.0))
            o_ref[0] = _dot_t(h, cls_ref[...]) + clsb_ref[...]

        head_block(i - 1, out1_ref)
        head_block(NB - i, out2_ref)


def _fwd_map(i):
    j = jnp.minimum(i, NB - 1)
    return (j, 0, 0)


def _bwd_map(i):
    return (NB - 1 - jnp.minimum(i, NB - 1), 0, 0)


def _yf_map(i):
    return (jnp.maximum(i - 1, 0), 0, 0)


def _yb_map(i):
    return (NB - 1 - jnp.maximum(i - 1, 0), 0, 0)


_Y_OUT = [
    jax.ShapeDtypeStruct((T, B, H), jnp.bfloat16),
    jax.ShapeDtypeStruct((T, B, H), jnp.bfloat16),
]

_SCRATCH = [
    pltpu.VMEM((2, UNROLL * B, G), jnp.bfloat16),
    pltpu.VMEM((2, UNROLL * B, G), jnp.bfloat16),
    pltpu.VMEM((B, H), jnp.float32),
    pltpu.VMEM((B, H), jnp.float32),
    pltpu.VMEM((B, H), jnp.float32),
    pltpu.VMEM((B, H), jnp.float32),
]


def _layer0(x0, wif, wib, whf, whb, b, len_col):
    return pl.pallas_call(
        _layer0_body,
        grid=(NB + 1,),
        in_specs=[
            pl.BlockSpec((B, 1), lambda i: (0, 0)),
            pl.BlockSpec((UNROLL, B, D_EMB), _fwd_map),
            pl.BlockSpec((UNROLL, B, D_EMB), _bwd_map),
            pl.BlockSpec((G, D_EMB), lambda i: (0, 0)),
            pl.BlockSpec((G, D_EMB), lambda i: (0, 0)),
            pl.BlockSpec((H, G), lambda i: (0, 0)),
            pl.BlockSpec((H, G), lambda i: (0, 0)),
            pl.BlockSpec((1, 2 * G), lambda i: (0, 0)),
        ],
        out_specs=[
            pl.BlockSpec((UNROLL, B, H), _yf_map),
            pl.BlockSpec((UNROLL, B, H), _yb_map),
        ],
        out_shape=_Y_OUT,
        scratch_shapes=list(_SCRATCH),
        compiler_params=pltpu.CompilerParams(
            dimension_semantics=("arbitrary",)),
    )(len_col, x0, x0, wif, wib, whf, whb, b.reshape(1, 2 * G))


NPAD = 128


def _o1_map(i):
    return (jnp.clip(i - 1, NB // 2, NB - 1) - NB // 2, 0, 0)


def _o2_map(i):
    return (jnp.clip(NB - i, 0, NB // 2 - 1), 0, 0)


def _layer1(yf0, yb0, wif, wib, whf, whb, b, fcw, fcb, cls_pad, clsb, len_col):
    return pl.pallas_call(
        _layer1_body,
        grid=(NB + 1,),
        in_specs=[
            pl.BlockSpec((B, 1), lambda i: (0, 0)),
            pl.BlockSpec((UNROLL, B, H), _fwd_map),
            pl.BlockSpec((UNROLL, B, H), _fwd_map),
            pl.BlockSpec((UNROLL, B, H), _bwd_map),
            pl.BlockSpec((UNROLL, B, H), _bwd_map),
            pl.BlockSpec((G, 2 * H), lambda i: (0, 0)),
            pl.BlockSpec((G, 2 * H), lambda i: (0, 0)),
            pl.BlockSpec((H, G), lambda i: (0, 0)),
            pl.BlockSpec((H, G), lambda i: (0, 0)),
            pl.BlockSpec((1, 2 * G), lambda i: (0, 0)),
            pl.BlockSpec((L_OUT, 2 * H), lambda i: (0, 0)),
            pl.BlockSpec((1, L_OUT), lambda i: (0, 0)),
            pl.BlockSpec((NPAD, L_OUT), lambda i: (0, 0)),
            pl.BlockSpec((1, NPAD), lambda i: (0, 0)),
        ],
        out_specs=[
            pl.BlockSpec((1, UNROLL * B, NPAD), _o1_map),
            pl.BlockSpec((1, UNROLL * B, NPAD), _o2_map),
        ],
        out_shape=[
            jax.ShapeDtypeStruct((NB // 2, UNROLL * B, NPAD), jnp.float32),
            jax.ShapeDtypeStruct((NB // 2, UNROLL * B, NPAD), jnp.float32),
        ],
        scratch_shapes=list(_SCRATCH) + [
            pltpu.VMEM((T, B, H), jnp.bfloat16),
            pltpu.VMEM((T, B, H), jnp.bfloat16),
        ],
        compiler_params=pltpu.CompilerParams(
            dimension_semantics=("arbitrary",)),
    )(len_col, yf0, yb0, yf0, yb0, wif, wib, whf, whb, b.reshape(1, 2 * G),
      fcw, fcb.reshape(1, L_OUT), cls_pad, clsb.reshape(1, NPAD))


def _head_body(ya_ref, yb_ref, w_ref, fcb_ref, cls_ref, clsb_ref, o_ref):
    h = (_dot_t(ya_ref[...], w_ref[:, :H]) + _dot_t(yb_ref[...], w_ref[:, H:])
         + fcb_ref[...])
    h = jnp.where(h > 0, h, 0.01 * (jnp.exp(jnp.minimum(h, 0.0)) - 1.0))
    o_ref[...] = _dot_t(h, cls_ref[...]) + clsb_ref[...]


def _head(ya, yb, w, fcb, cls_pad, clsb, bm):
    m = ya.shape[0]
    n = cls_pad.shape[0]
    return pl.pallas_call(
        _head_body,
        grid=(m // bm,),
        in_specs=[
            pl.BlockSpec((bm, H), lambda i: (i, 0)),
            pl.BlockSpec((bm, H), lambda i: (i, 0)),
            pl.BlockSpec((L_OUT, 2 * H), lambda i: (0, 0)),
            pl.BlockSpec((1, L_OUT), lambda i: (0, 0)),
            pl.BlockSpec((n, L_OUT), lambda i: (0, 0)),
            pl.BlockSpec((1, n), lambda i: (0, 0)),
        ],
        out_specs=pl.BlockSpec((bm, n), lambda i: (i, 0)),
        out_shape=jax.ShapeDtypeStruct((m, n), jnp.float32),
    )(ya, yb, w, fcb.reshape(1, L_OUT), cls_pad, clsb.reshape(1, n))


def kernel(inputs, lengths, emb, Wih_l0f, Whh_l0f, bih_l0f, bhh_l0f,
           Wih_l0b, Whh_l0b, bih_l0b, bhh_l0b,
           Wih_l1f, Whh_l1f, bih_l1f, bhh_l1f,
           Wih_l1b, Whh_l1b, bih_l1b, bhh_l1b,
           fc_w, fc_b, cls_w, cls_b):
    f32 = jnp.float32
    bf16 = jnp.bfloat16
    idx = inputs.T.reshape(N).astype(jnp.int32)
    len_col = lengths.astype(f32).reshape(B, 1)

    b0 = jnp.concatenate([bih_l0f + bhh_l0f, bih_l0b + bhh_l0b])
    b1 = jnp.concatenate([bih_l1f + bhh_l1f, bih_l1b + bhh_l1b])
    n_pad = 128
    cls_pad = jnp.zeros((n_pad, L_OUT), f32).at[:TAGS].set(cls_w)
    clsb_pad = jnp.zeros((n_pad,), f32).at[:TAGS].set(cls_b)

    x0 = _sc_gather(emb, idx).reshape(T, B, D_EMB)

    yf0, yb0 = _layer0(x0, Wih_l0f.astype(bf16), Wih_l0b.astype(bf16),
                       Whh_l0f.T.astype(bf16), Whh_l0b.T.astype(bf16),
                       b0, len_col)

    out_hi, out_lo = _layer1(yf0, yb0, Wih_l1f.astype(bf16), Wih_l1b.astype(bf16),
                             Whh_l1f.T.astype(bf16), Whh_l1b.T.astype(bf16),
                             b1, fc_w.astype(bf16), fc_b, cls_pad, clsb_pad,
                             len_col)
    out = jnp.concatenate([out_lo.reshape(N // 2, n_pad),
                           out_hi.reshape(N // 2, n_pad)], axis=0)
    return out.reshape(T, B, n_pad).transpose(1, 0, 2)[:, :, :TAGS]


# speedup vs baseline: 1.0133x; 1.0133x over previous
import functools

import jax
import jax.numpy as jnp
from jax.experimental import pallas as pl
from jax.experimental.pallas import tpu as pltpu
from jax.experimental.pallas import tpu_sc as plsc

B, T, VOCAB, D_EMB, H, L_OUT, TAGS = 64, 128, 100000, 256, 512, 256, 50
G = 4 * H
N = T * B
UNROLL = 16
NB = T // UNROLL

_GATHER_WINDOW = 128


def _sc_gather(emb, idx):
    mesh = plsc.VectorSubcoreMesh(core_axis_name="core", subcore_axis_name="subcore")

    @pl.kernel(out_type=jax.ShapeDtypeStruct((N, D_EMB), emb.dtype), mesh=mesh)
    def gather_kernel(emb_hbm, i_hbm, o_hbm):
        def body(i_vmem, o_vmem):
            pltpu.sync_copy(emb_hbm.at[i_vmem.at[0]], o_vmem)

        pltpu.emit_pipeline(
            body,
            grid=(N // _GATHER_WINDOW,),
            in_specs=[pl.BlockSpec((1, _GATHER_WINDOW), index_map=lambda i: (0, i))],
            out_specs=[pl.BlockSpec((_GATHER_WINDOW, D_EMB), index_map=lambda i: (i, 0))],
            core_axis_name=("core", "subcore"),
            dimension_semantics=(pltpu.PARALLEL,),
        )(i_hbm, o_hbm)

    return gather_kernel(emb, idx.reshape(1, N))


def _dot(a, b):
    return jax.lax.dot_general(a, b, (((1,), (0,)), ((), ())),
                               preferred_element_type=jnp.float32)


def _dot_t(a, b):
    return jax.lax.dot_general(a, b, (((1,), (1,)), ((), ())),
                               preferred_element_type=jnp.float32)


def _recurrence_block(i, len_ref, sxf, sxb, yf_ref, yb_ref, hf, cf, hb, cb,
                      whf_ref, whb_ref):
    @pl.when(i == 1)
    def _():
        zero = jnp.zeros((B, H), jnp.float32)
        hf[...] = zero
        cf[...] = zero
        hb[...] = zero
        cb[...] = zero

    lens = len_ref[...]
    buf = (i - 1) % 2

    def gates(g, c):
        gi = jax.nn.sigmoid(g[:, :H])
        gf = jax.nn.sigmoid(g[:, H:2 * H])
        gg = jnp.tanh(g[:, 2 * H:3 * H])
        go = jax.nn.sigmoid(g[:, 3 * H:])
        c_new = gf * c + gi * gg
        h_new = go * jnp.tanh(c_new)
        return h_new, c_new

    t0 = (i - 1) * UNROLL
    for k in range(UNROLL):
        tf = t0 + k
        tb = T - 1 - tf
        g_f = sxf[buf, pl.ds(B * k, B), :] + _dot(hf[...].astype(jnp.bfloat16),
                                                  whf_ref[...])
        g_b = (sxb[buf, pl.ds(B * (UNROLL - 1 - k), B), :]
               + _dot(hb[...].astype(jnp.bfloat16), whb_ref[...]))
        hn_f, cn_f = gates(g_f, cf[...])
        hn_b, cn_b = gates(g_b, cb[...])
        m_f = (lens > jnp.float32(tf)).astype(jnp.float32)
        m_b = (lens > jnp.float32(tb)).astype(jnp.float32)
        yf_ref[k] = (hn_f * m_f).astype(yf_ref.dtype)
        yb_ref[UNROLL - 1 - k] = (hn_b * m_b).astype(yb_ref.dtype)
        hf[...] = m_f * hn_f + (1.0 - m_f) * hf[...]
        cf[...] = m_f * cn_f + (1.0 - m_f) * cf[...]
        hb[...] = m_b * hn_b + (1.0 - m_b) * hb[...]
        cb[...] = m_b * cn_b + (1.0 - m_b) * cb[...]


def _layer0_body(len_ref, pf_ref, pb_ref, wif_ref, wib_ref, whf_ref, whb_ref,
                 b_ref, yf_ref, yb_ref, sxf, sxb, hf, cf, hb, cb):
    i = pl.program_id(0)

    @pl.when(i < NB)
    def _():
        xf = pf_ref[...].reshape(UNROLL * B, D_EMB).astype(jnp.bfloat16)
        xb = pb_ref[...].reshape(UNROLL * B, D_EMB).astype(jnp.bfloat16)
        sxf[i % 2] = (_dot_t(xf, wif_ref[...]) + b_ref[:, :G]).astype(sxf.dtype)
        sxb[i % 2] = (_dot_t(xb, wib_ref[...]) + b_ref[:, G:]).astype(sxb.dtype)

    @pl.when(i > 0)
    def _():
        _recurrence_block(i, len_ref, sxf, sxb, yf_ref, yb_ref,
                          hf, cf, hb, cb, whf_ref, whb_ref)


def _layer1_body(len_ref, pfa_ref, pfb_ref, pba_ref, pbb_ref,
                 wif_ref, wib_ref, whf_ref, whb_ref, b_ref,
                 yf_ref, yb_ref, sxf, sxb, hf, cf, hb, cb):
    i = pl.program_id(0)

    @pl.when(i < NB)
    def _():
        ya_f = pfa_ref[...].reshape(UNROLL * B, H)
        yb_f = pfb_ref[...].reshape(UNROLL * B, H)
        ya_b = pba_ref[...].reshape(UNROLL * B, H)
        yb_b = pbb_ref[...].reshape(UNROLL * B, H)
        sxf[i % 2] = (_dot_t(ya_f, wif_ref[:, :H]) + _dot_t(yb_f, wif_ref[:, H:])
                      + b_ref[:, :G]).astype(sxf.dtype)
        sxb[i % 2] = (_dot_t(ya_b, wib_ref[:, :H]) + _dot_t(yb_b, wib_ref[:, H:])
                      + b_ref[:, G:]).astype(sxb.dtype)

    @pl.when(i > 0)
    def _():
        _recurrence_block(i, len_ref, sxf, sxb, yf_ref, yb_ref,
                          hf, cf, hb, cb, whf_ref, whb_ref)


def _fwd_map(i):
    j = jnp.minimum(i, NB - 1)
    return (j, 0, 0)


def _bwd_map(i):
    return (NB - 1 - jnp.minimum(i, NB - 1), 0, 0)


def _yf_map(i):
    return (jnp.maximum(i - 1, 0), 0, 0)


def _yb_map(i):
    return (NB - 1 - jnp.maximum(i - 1, 0), 0, 0)


_Y_OUT = [
    jax.ShapeDtypeStruct((T, B, H), jnp.bfloat16),
    jax.ShapeDtypeStruct((T, B, H), jnp.bfloat16),
]

_SCRATCH = [
    pltpu.VMEM((2, UNROLL * B, G), jnp.bfloat16),
    pltpu.VMEM((2, UNROLL * B, G), jnp.bfloat16),
    pltpu.VMEM((B, H), jnp.float32),
    pltpu.VMEM((B, H), jnp.float32),
    pltpu.VMEM((B, H), jnp.float32),
    pltpu.VMEM((B, H), jnp.float32),
]


def _layer0(x0, wif, wib, whf, whb, b, len_col):
    return pl.pallas_call(
        _layer0_body,
        grid=(NB + 1,),
        in_specs=[
            pl.BlockSpec((B, 1), lambda i: (0, 0)),
            pl.BlockSpec((UNROLL, B, D_EMB), _fwd_map),
            pl.BlockSpec((UNROLL, B, D_EMB), _bwd_map),
            pl.BlockSpec((G, D_EMB), lambda i: (0, 0)),
            pl.BlockSpec((G, D_EMB), lambda i: (0, 0)),
            pl.BlockSpec((H, G), lambda i: (0, 0)),
            pl.BlockSpec((H, G), lambda i: (0, 0)),
            pl.BlockSpec((1, 2 * G), lambda i: (0, 0)),
        ],
        out_specs=[
            pl.BlockSpec((UNROLL, B, H), _yf_map),
            pl.BlockSpec((UNROLL, B, H), _yb_map),
        ],
        out_shape=_Y_OUT,
        scratch_shapes=list(_SCRATCH),
        compiler_params=pltpu.CompilerParams(
            dimension_semantics=("arbitrary",)),
    )(len_col, x0, x0, wif, wib, whf, whb, b.reshape(1, 2 * G))


def _layer1(yf0, yb0, wif, wib, whf, whb, b, len_col):
    return pl.pallas_call(
        _layer1_body,
        grid=(NB + 1,),
        in_specs=[
            pl.BlockSpec((B, 1), lambda i: (0, 0)),
            pl.BlockSpec((UNROLL, B, H), _fwd_map),
            pl.BlockSpec((UNROLL, B, H), _fwd_map),
            pl.BlockSpec((UNROLL, B, H), _bwd_map),
            pl.BlockSpec((UNROLL, B, H), _bwd_map),
            pl.BlockSpec((G, 2 * H), lambda i: (0, 0)),
            pl.BlockSpec((G, 2 * H), lambda i: (0, 0)),
            pl.BlockSpec((H, G), lambda i: (0, 0)),
            pl.BlockSpec((H, G), lambda i: (0, 0)),
            pl.BlockSpec((1, 2 * G), lambda i: (0, 0)),
        ],
        out_specs=[
            pl.BlockSpec((UNROLL, B, H), _yf_map),
            pl.BlockSpec((UNROLL, B, H), _yb_map),
        ],
        out_shape=_Y_OUT,
        scratch_shapes=list(_SCRATCH),
        compiler_params=pltpu.CompilerParams(
            dimension_semantics=("arbitrary",)),
    )(len_col, yf0, yb0, yf0, yb0, wif, wib, whf, whb, b.reshape(1, 2 * G))


def _head_body(ya_ref, yb_ref, w_ref, fcb_ref, cls_ref, clsb_ref, o_ref):
    h = (_dot_t(ya_ref[...], w_ref[:, :H]) + _dot_t(yb_ref[...], w_ref[:, H:])
         + fcb_ref[...])
    h = jnp.where(h > 0, h, 0.01 * (jnp.exp(jnp.minimum(h, 0.0)) - 1.0))
    o_ref[...] = _dot_t(h, cls_ref[...]) + clsb_ref[...]


def _head(ya, yb, w, fcb, cls_pad, clsb, bm):
    m = ya.shape[0]
    n = cls_pad.shape[0]
    return pl.pallas_call(
        _head_body,
        grid=(m // bm,),
        in_specs=[
            pl.BlockSpec((bm, H), lambda i: (i, 0)),
            pl.BlockSpec((bm, H), lambda i: (i, 0)),
            pl.BlockSpec((L_OUT, 2 * H), lambda i: (0, 0)),
            pl.BlockSpec((1, L_OUT), lambda i: (0, 0)),
            pl.BlockSpec((n, L_OUT), lambda i: (0, 0)),
            pl.BlockSpec((1, n), lambda i: (0, 0)),
        ],
        out_specs=pl.BlockSpec((bm, n), lambda i: (i, 0)),
        out_shape=jax.ShapeDtypeStruct((m, n), jnp.float32),
    )(ya, yb, w, fcb.reshape(1, L_OUT), cls_pad, clsb.reshape(1, n))


def kernel(inputs, lengths, emb, Wih_l0f, Whh_l0f, bih_l0f, bhh_l0f,
           Wih_l0b, Whh_l0b, bih_l0b, bhh_l0b,
           Wih_l1f, Whh_l1f, bih_l1f, bhh_l1f,
           Wih_l1b, Whh_l1b, bih_l1b, bhh_l1b,
           fc_w, fc_b, cls_w, cls_b):
    f32 = jnp.float32
    bf16 = jnp.bfloat16
    idx = inputs.T.reshape(N).astype(jnp.int32)
    len_col = lengths.astype(f32).reshape(B, 1)

    b0 = jnp.concatenate([bih_l0f + bhh_l0f, bih_l0b + bhh_l0b])
    b1 = jnp.concatenate([bih_l1f + bhh_l1f, bih_l1b + bhh_l1b])
    n_pad = 128
    cls_pad = jnp.zeros((n_pad, L_OUT), f32).at[:TAGS].set(cls_w)
    clsb_pad = jnp.zeros((n_pad,), f32).at[:TAGS].set(cls_b)

    x0 = _sc_gather(emb, idx).reshape(T, B, D_EMB)

    yf0, yb0 = _layer0(x0, Wih_l0f.astype(bf16), Wih_l0b.astype(bf16),
                       Whh_l0f.T.astype(bf16), Whh_l0b.T.astype(bf16),
                       b0, len_col)

    yf1, yb1 = _layer1(yf0, yb0, Wih_l1f.astype(bf16), Wih_l1b.astype(bf16),
                       Whh_l1f.T.astype(bf16), Whh_l1b.T.astype(bf16),
                       b1, len_col)

    out = _head(yf1.reshape(N, H), yb1.reshape(N, H), fc_w.astype(bf16),
                fc_b, cls_pad, clsb_pad, bm=1024)
    return out.reshape(T, B, n_pad).transpose(1, 0, 2)[:, :, :TAGS]
